# baseline (speedup 1.0000x reference)
import functools

import jax
import jax.numpy as jnp
from jax import lax
from jax.experimental import pallas as pl
from jax.experimental.pallas import tpu as pltpu
from jax.experimental.pallas import tpu_sc as plsc

_SIZE = 128
_BATCH = 16384
_NC = 2
_NS = 16
_NW = _NC * _NS
_BPW = _BATCH // _NW
_CH = 128
_NCHUNK = _BPW // _CH

_mesh = plsc.VectorSubcoreMesh(core_axis_name="c", subcore_axis_name="s")


@functools.partial(
    pl.kernel,
    mesh=_mesh,
    out_type=jax.ShapeDtypeStruct((_BATCH, _SIZE), jnp.float32),
    scratch_types=[
        pltpu.VMEM((_NCHUNK, _CH), jnp.int32),
        pltpu.VMEM((_BPW, _SIZE), jnp.float32),
        pltpu.SemaphoreType.DMA((_NCHUNK,)),
        pltpu.SemaphoreType.DMA((_NCHUNK,)),
        pltpu.SemaphoreType.DMA,
    ],
)
def _pe_gather(idx_hbm, pe_hbm, out_hbm, idx_v, rows_v, isems, gsems, ssem):
    wid = lax.axis_index("s") * _NC + lax.axis_index("c")
    base = wid * _BPW
    idx_copies = [
        pltpu.async_copy(
            idx_hbm.at[wid * _NCHUNK + j], idx_v.at[j], isems.at[j]
        )
        for j in range(_NCHUNK)
    ]
    gathers = []
    for j in range(_NCHUNK):
        idx_copies[j].wait()
        gathers.append(
            pltpu.async_copy(
                pe_hbm.at[idx_v.at[j]], rows_v.at[pl.ds(j * _CH, _CH)], gsems.at[j]
            )
        )
    stores = []
    for j in range(_NCHUNK):
        gathers[j].wait()
        stores.append(
            pltpu.async_copy(
                rows_v.at[pl.ds(j * _CH, _CH)],
                out_hbm.at[pl.ds(base + j * _CH, _CH)],
                ssem,
            )
        )
    for s in stores:
        s.wait()


def kernel(t, pe):
    idx2 = t.reshape(_NW * _NCHUNK, _CH)
    return _pe_gather(idx2, pe)


# speedup vs baseline: 1.0173x; 1.0173x over previous
import functools

import jax
import jax.numpy as jnp
from jax import lax
from jax.experimental import pallas as pl
from jax.experimental.pallas import tpu as pltpu
from jax.experimental.pallas import tpu_sc as plsc

_SIZE = 128
_BATCH = 16384
_NC = 2
_NS = 16
_NW = _NC * _NS
_BPW = _BATCH // _NW
_CH = 128
_NCHUNK = _BPW // _CH

_mesh = plsc.VectorSubcoreMesh(core_axis_name="c", subcore_axis_name="s")


@functools.partial(
    pl.kernel,
    mesh=_mesh,
    out_type=jax.ShapeDtypeStruct((_BATCH, _SIZE), jnp.float32),
    scratch_types=[
        pltpu.VMEM((_NCHUNK, _CH), jnp.int32),
        pltpu.VMEM((_BPW, _SIZE), jnp.float32),
        pltpu.SemaphoreType.DMA,
    ],
)
def _pe_gather(idx_hbm, pe_hbm, out_hbm, idx_v, rows_v, sem):
    wid = lax.axis_index("s") * _NC + lax.axis_index("c")
    base = wid * _BPW
    pltpu.sync_copy(idx_hbm.at[wid], idx_v)
    copies = [
        pltpu.async_copy(
            pe_hbm.at[idx_v.at[j]], rows_v.at[pl.ds(j * _CH, _CH)], sem
        )
        for j in range(_NCHUNK)
    ]
    for c in copies:
        c.wait()
    pltpu.sync_copy(rows_v, out_hbm.at[pl.ds(base, _BPW)])


def kernel(t, pe):
    idx3 = t.reshape(_NW, _NCHUNK, _CH)
    return _pe_gather(idx3, pe)
